# Initial kernel scaffold; baseline (speedup 1.0000x reference)
import jax
import jax.numpy as jnp
from jax.experimental import pallas as pl


def kernel(x, edge_index, W1, att_src1, att_dst1, b1, W2, att_src2, att_dst2, b2):
    raise NotImplementedError("write your pallas kernel here")


# speedup vs baseline: 44.7534x; 44.7534x over previous
import functools

import jax
import jax.numpy as jnp
from jax import lax
from jax.experimental import pallas as pl
from jax.experimental.pallas import tpu as pltpu
from jax.experimental.pallas import tpu_sc as plsc

N = 10000
E = 320000
D_IN = 128
F = 64
AW = 16
PK = AW + F

NC = 2
NS = 16
NW = NC * NS
EPW = E // NW
CHUNK = 80
NCHUNK = EPW // CHUNK
NP = 10240
RPT = NP // NS

BLK = 400
GRID = N // BLK


def _pre_body(x_ref, w_ref, am_ref, bm_ref, s_out, ad_out):
    h = jnp.dot(x_ref[:], w_ref[:], preferred_element_type=jnp.float32)
    asrc = jnp.dot(h, am_ref[:], preferred_element_type=jnp.float32)
    adst = jnp.dot(h, bm_ref[:], preferred_element_type=jnp.float32)
    s_out[:] = jnp.concatenate([asrc, h], axis=1)
    ad_out[:] = adst


def _pre_call(x, W, am, bm, d_in):
    return pl.pallas_call(
        _pre_body,
        grid=(GRID,),
        in_specs=[
            pl.BlockSpec((BLK, d_in), lambda i: (i, 0)),
            pl.BlockSpec((d_in, F), lambda i: (0, 0)),
            pl.BlockSpec((F, AW), lambda i: (0, 0)),
            pl.BlockSpec((F, AW), lambda i: (0, 0)),
        ],
        out_specs=[
            pl.BlockSpec((BLK, PK), lambda i: (i, 0)),
            pl.BlockSpec((BLK, AW), lambda i: (i, 0)),
        ],
        out_shape=[
            jax.ShapeDtypeStruct((N, PK), jnp.float32),
            jax.ShapeDtypeStruct((N, AW), jnp.float32),
        ],
    )(x, W, am, bm)


def _post_body(s_ref, ad_ref, p0_ref, p1_ref, b_ref, r_ref, o_ref):
    S = s_ref[:]
    a16 = S[:, 0:AW]
    h = S[:, AW:PK]
    al = a16 + ad_ref[:]
    ps = jnp.exp(jnp.maximum(al, 0.2 * al))
    den16 = ps + p0_ref[:, 0:AW] + p1_ref[:, 0:AW]
    R = r_ref[:]
    msg = (jnp.dot(ps, R, preferred_element_type=jnp.float32) * h
           + p0_ref[:, AW:PK] + p1_ref[:, AW:PK])
    den = jnp.dot(den16, R, preferred_element_type=jnp.float32)
    out = msg / (den + 1e-16) + b_ref[:]
    o_ref[:] = jnp.where(out > 0, out, jnp.exp(jnp.minimum(out, 0.0)) - 1.0)


def _post_call(S, Ad, P0, P1, b, R):
    return pl.pallas_call(
        _post_body,
        grid=(GRID,),
        in_specs=[
            pl.BlockSpec((BLK, PK), lambda i: (i, 0)),
            pl.BlockSpec((BLK, AW), lambda i: (i, 0)),
            pl.BlockSpec((BLK, PK), lambda i: (i, 0)),
            pl.BlockSpec((BLK, PK), lambda i: (i, 0)),
            pl.BlockSpec((1, F), lambda i: (0, 0)),
            pl.BlockSpec((AW, F), lambda i: (0, 0)),
        ],
        out_specs=pl.BlockSpec((BLK, F), lambda i: (i, 0)),
        out_shape=jax.ShapeDtypeStruct((N, F), jnp.float32),
    )(S, Ad, P0, P1, b, R)


def _vgather(vec, idx):
    dn = lax.GatherDimensionNumbers(
        offset_dims=(), collapsed_slice_dims=(0,), start_index_map=(0,))
    return lax.gather(vec, idx[:, None], dn, slice_sizes=(1,),
                      mode=lax.GatherScatterMode.PROMISE_IN_BOUNDS)


def _make_edge(heads):
    mesh = plsc.VectorSubcoreMesh(core_axis_name="c", subcore_axis_name="s")

    @functools.partial(
        pl.kernel,
        mesh=mesh,
        out_type=[
            jax.ShapeDtypeStruct((NP, PK), jnp.float32),
            jax.ShapeDtypeStruct((NP, PK), jnp.float32),
        ],
        scratch_types=[
            pltpu.VMEM((CHUNK,), jnp.int32),
            pltpu.VMEM((CHUNK,), jnp.int32),
            pltpu.VMEM((CHUNK, PK), jnp.float32),
            pltpu.VMEM((CHUNK, AW), jnp.float32),
            pltpu.VMEM_SHARED((NP, PK), jnp.float32),
            pltpu.SemaphoreType.DMA,
            pltpu.SemaphoreType.DMA,
        ],
        compiler_params=pltpu.CompilerParams(use_tc_tiling_on_sc=False),
    )
    def edge_kernel(s_hbm, ad_hbm, src_hbm, dst_hbm, zero_hbm,
                    out0, out1, src_v, dst_v, srow, arow, acc, g1, g2):
        c = lax.axis_index("c")
        s = lax.axis_index("s")
        wid = s * NC + c
        pltpu.sync_copy(zero_hbm.at[pl.ds(s * RPT, RPT)],
                        acc.at[pl.ds(s * RPT, RPT)])
        plsc.subcore_barrier()
        lane = lax.iota(jnp.int32, 16)
        base = wid * EPW

        def chunk_body(i, carry):
            off = base + i * CHUNK
            pltpu.sync_copy(src_hbm.at[pl.ds(off, CHUNK)], src_v)
            pltpu.sync_copy(dst_hbm.at[pl.ds(off, CHUNK)], dst_v)
            pltpu.async_copy(s_hbm.at[src_v], srow, g1).wait()
            pltpu.async_copy(ad_hbm.at[dst_v], arow, g2).wait()

            def edge_body(e, c2):
                va = srow[e, pl.ds(0, AW)]
                vb = arow[e, pl.ds(0, AW)]
                al = va + vb
                p = jnp.exp(jnp.maximum(al, 0.2 * al))
                srow[e, pl.ds(0, AW)] = p
                for k in range(1, 5):
                    if heads == 8:
                        col_idx = (lane + (16 * (k - 1))) >> 3
                    else:
                        col_idx = lane * 0
                    hv = srow[e, pl.ds(16 * k, 16)]
                    pm = _vgather(p, col_idx)
                    srow[e, pl.ds(16 * k, 16)] = hv * pm
                return c2

            lax.fori_loop(0, CHUNK, edge_body, 0)
            pltpu.sync_copy(srow, acc.at[dst_v], add=True)
            return carry

        lax.fori_loop(0, NCHUNK, chunk_body, 0)
        plsc.subcore_barrier()

        @pl.when(c == 0)
        def _():
            pltpu.sync_copy(acc.at[pl.ds(s * RPT, RPT)],
                            out0.at[pl.ds(s * RPT, RPT)])

        @pl.when(c == 1)
        def _():
            pltpu.sync_copy(acc.at[pl.ds(s * RPT, RPT)],
                            out1.at[pl.ds(s * RPT, RPT)])

    return edge_kernel


_edge8 = _make_edge(8)
_edge1 = _make_edge(1)


def _placement(att, heads, ch):
    a = att.reshape(heads * ch).astype(jnp.float32)
    rows = jnp.arange(F)
    return jnp.zeros((F, AW), jnp.float32).at[rows, rows // ch].set(a)


def _rep_matrix(ch):
    col_head = jnp.arange(F) // ch
    return (col_head[None, :] == jnp.arange(AW)[:, None]).astype(jnp.float32)


def kernel(x, edge_index, W1, att_src1, att_dst1, b1,
           W2, att_src2, att_dst2, b2):
    src = edge_index[0].astype(jnp.int32)
    dst = edge_index[1].astype(jnp.int32)
    zeros_pk = jnp.zeros((NP, PK), jnp.float32)
    am1 = _placement(att_src1, 8, 8)
    bm1 = _placement(att_dst1, 8, 8)
    am2 = _placement(att_src2, 1, 64)
    bm2 = _placement(att_dst2, 1, 64)
    R1 = _rep_matrix(8)
    R2 = _rep_matrix(64)

    S1, Ad1 = _pre_call(x, W1.astype(jnp.float32), am1, bm1, D_IN)
    P0, P1 = _edge8(S1, Ad1, src, dst, zeros_pk)
    h1 = _post_call(S1, Ad1, P0, P1, b1.reshape(1, F), R1)

    S2, Ad2 = _pre_call(h1, W2.astype(jnp.float32), am2, bm2, F)
    Q0, Q1 = _edge1(S2, Ad2, src, dst, zeros_pk)
    return _post_call(S2, Ad2, Q0, Q1, b2.reshape(1, F), R2)


# speedup vs baseline: 88.2151x; 1.9711x over previous
import functools

import jax
import jax.numpy as jnp
from jax import lax
from jax.experimental import pallas as pl
from jax.experimental.pallas import tpu as pltpu
from jax.experimental.pallas import tpu_sc as plsc

N = 10000
E = 320000
D_IN = 128
F = 64
AW = 16
PK = AW + F

NC = 2
NS = 16
NW = NC * NS
EPW = E // NW
CHUNK = 80
NCHUNK = EPW // CHUNK
NP = 10240
RPT = NP // NS

BLK = 400
GRID = N // BLK


def _pre_body(x_ref, w_ref, am_ref, bm_ref, s_out, ad_out):
    h = jnp.dot(x_ref[:], w_ref[:], preferred_element_type=jnp.float32)
    asrc = jnp.dot(h, am_ref[:], preferred_element_type=jnp.float32)
    adst = jnp.dot(h, bm_ref[:], preferred_element_type=jnp.float32)
    s_out[:] = jnp.concatenate([asrc, h], axis=1)
    ad_out[:] = adst


def _pre_call(x, W, am, bm, d_in):
    return pl.pallas_call(
        _pre_body,
        grid=(GRID,),
        in_specs=[
            pl.BlockSpec((BLK, d_in), lambda i: (i, 0)),
            pl.BlockSpec((d_in, F), lambda i: (0, 0)),
            pl.BlockSpec((F, AW), lambda i: (0, 0)),
            pl.BlockSpec((F, AW), lambda i: (0, 0)),
        ],
        out_specs=[
            pl.BlockSpec((BLK, PK), lambda i: (i, 0)),
            pl.BlockSpec((BLK, AW), lambda i: (i, 0)),
        ],
        out_shape=[
            jax.ShapeDtypeStruct((N, PK), jnp.float32),
            jax.ShapeDtypeStruct((N, AW), jnp.float32),
        ],
    )(x, W, am, bm)


def _post_body(s_ref, ad_ref, p0_ref, p1_ref, b_ref, r_ref, o_ref):
    S = s_ref[:]
    a16 = S[:, 0:AW]
    h = S[:, AW:PK]
    al = a16 + ad_ref[:]
    ps = jnp.exp(jnp.maximum(al, 0.2 * al))
    den16 = ps + p0_ref[:, 0:AW] + p1_ref[:, 0:AW]
    R = r_ref[:]
    msg = (jnp.dot(ps, R, preferred_element_type=jnp.float32) * h
           + p0_ref[:, AW:PK] + p1_ref[:, AW:PK])
    den = jnp.dot(den16, R, preferred_element_type=jnp.float32)
    out = msg / (den + 1e-16) + b_ref[:]
    o_ref[:] = jnp.where(out > 0, out, jnp.exp(jnp.minimum(out, 0.0)) - 1.0)


def _post_call(S, Ad, P0, P1, b, R):
    return pl.pallas_call(
        _post_body,
        grid=(GRID,),
        in_specs=[
            pl.BlockSpec((BLK, PK), lambda i: (i, 0)),
            pl.BlockSpec((BLK, AW), lambda i: (i, 0)),
            pl.BlockSpec((BLK, PK), lambda i: (i, 0)),
            pl.BlockSpec((BLK, PK), lambda i: (i, 0)),
            pl.BlockSpec((1, F), lambda i: (0, 0)),
            pl.BlockSpec((AW, F), lambda i: (0, 0)),
        ],
        out_specs=pl.BlockSpec((BLK, F), lambda i: (i, 0)),
        out_shape=jax.ShapeDtypeStruct((N, F), jnp.float32),
    )(S, Ad, P0, P1, b, R)


def _vgather(vec, idx):
    dn = lax.GatherDimensionNumbers(
        offset_dims=(), collapsed_slice_dims=(0,), start_index_map=(0,))
    return lax.gather(vec, idx[:, None], dn, slice_sizes=(1,),
                      mode=lax.GatherScatterMode.PROMISE_IN_BOUNDS)


def _make_edge(heads):
    mesh = plsc.VectorSubcoreMesh(core_axis_name="c", subcore_axis_name="s")

    @functools.partial(
        pl.kernel,
        mesh=mesh,
        out_type=[
            jax.ShapeDtypeStruct((NP, PK), jnp.float32),
            jax.ShapeDtypeStruct((NP, PK), jnp.float32),
        ],
        scratch_types=[
            pltpu.VMEM((NCHUNK, CHUNK), jnp.int32),
            pltpu.VMEM((NCHUNK, CHUNK), jnp.int32),
            pltpu.VMEM((CHUNK, PK), jnp.float32),
            pltpu.VMEM((CHUNK, PK), jnp.float32),
            pltpu.VMEM((CHUNK, AW), jnp.float32),
            pltpu.VMEM((CHUNK, AW), jnp.float32),
            pltpu.VMEM_SHARED((NP, PK), jnp.float32),
            pltpu.SemaphoreType.DMA,
            pltpu.SemaphoreType.DMA,
            pltpu.SemaphoreType.DMA,
            pltpu.SemaphoreType.DMA,
        ],
        compiler_params=pltpu.CompilerParams(use_tc_tiling_on_sc=False),
    )
    def edge_kernel(s_hbm, ad_hbm, src_hbm, dst_hbm, zero_hbm,
                    out0, out1, src_all, dst_all, srow0, srow1,
                    arow0, arow1, acc, ga0, gb0, ga1, gb1):
        c = lax.axis_index("c")
        s = lax.axis_index("s")
        wid = s * NC + c
        pltpu.sync_copy(zero_hbm.at[pl.ds(s * RPT, RPT)],
                        acc.at[pl.ds(s * RPT, RPT)])
        pltpu.sync_copy(src_hbm.at[pl.ds(wid * NCHUNK, NCHUNK)], src_all)
        pltpu.sync_copy(dst_hbm.at[pl.ds(wid * NCHUNK, NCHUNK)], dst_all)
        plsc.subcore_barrier()
        lane = lax.iota(jnp.int32, 16)

        def start(i, srow, arow, g1, g2):
            pltpu.async_copy(s_hbm.at[src_all.at[i]], srow, g1)
            pltpu.async_copy(ad_hbm.at[dst_all.at[i]], arow, g2)

        def process(i, srow, arow, g1, g2):
            pltpu.make_async_copy(s_hbm.at[src_all.at[i]], srow, g1).wait()
            pltpu.make_async_copy(ad_hbm.at[dst_all.at[i]], arow, g2).wait()

            def edge_body(e, c2):
                va = srow[e, pl.ds(0, AW)]
                vb = arow[e, pl.ds(0, AW)]
                al = va + vb
                p = jnp.exp(jnp.maximum(al, 0.2 * al))
                srow[e, pl.ds(0, AW)] = p
                for k in range(1, 5):
                    if heads == 8:
                        col_idx = (lane + (16 * (k - 1))) >> 3
                    else:
                        col_idx = lane * 0
                    hv = srow[e, pl.ds(16 * k, 16)]
                    pm = _vgather(p, col_idx)
                    srow[e, pl.ds(16 * k, 16)] = hv * pm
                return c2

            lax.fori_loop(0, CHUNK, edge_body, 0)
            pltpu.sync_copy(srow, acc.at[dst_all.at[i]], add=True)

        start(0, srow0, arow0, ga0, gb0)

        def pair_body(j, carry):
            start(2 * j + 1, srow1, arow1, ga1, gb1)
            process(2 * j, srow0, arow0, ga0, gb0)
            start(2 * j + 2, srow0, arow0, ga0, gb0)
            process(2 * j + 1, srow1, arow1, ga1, gb1)
            return carry

        lax.fori_loop(0, (NCHUNK - 1) // 2, pair_body, 0)
        process(NCHUNK - 1, srow0, arow0, ga0, gb0)
        plsc.subcore_barrier()

        @pl.when(c == 0)
        def _():
            pltpu.sync_copy(acc.at[pl.ds(s * RPT, RPT)],
                            out0.at[pl.ds(s * RPT, RPT)])

        @pl.when(c == 1)
        def _():
            pltpu.sync_copy(acc.at[pl.ds(s * RPT, RPT)],
                            out1.at[pl.ds(s * RPT, RPT)])

    return edge_kernel


_edge8 = _make_edge(8)
_edge1 = _make_edge(1)


def _placement(att, heads, ch):
    a = att.reshape(heads * ch).astype(jnp.float32)
    rows = jnp.arange(F)
    return jnp.zeros((F, AW), jnp.float32).at[rows, rows // ch].set(a)


def _rep_matrix(ch):
    col_head = jnp.arange(F) // ch
    return (col_head[None, :] == jnp.arange(AW)[:, None]).astype(jnp.float32)


def kernel(x, edge_index, W1, att_src1, att_dst1, b1,
           W2, att_src2, att_dst2, b2):
    src = edge_index[0].astype(jnp.int32).reshape(E // CHUNK, CHUNK)
    dst = edge_index[1].astype(jnp.int32).reshape(E // CHUNK, CHUNK)
    zeros_pk = jnp.zeros((NP, PK), jnp.float32)
    am1 = _placement(att_src1, 8, 8)
    bm1 = _placement(att_dst1, 8, 8)
    am2 = _placement(att_src2, 1, 64)
    bm2 = _placement(att_dst2, 1, 64)
    R1 = _rep_matrix(8)
    R2 = _rep_matrix(64)

    S1, Ad1 = _pre_call(x, W1.astype(jnp.float32), am1, bm1, D_IN)
    P0, P1 = _edge8(S1, Ad1, src, dst, zeros_pk)
    h1 = _post_call(S1, Ad1, P0, P1, b1.reshape(1, F), R1)

    S2, Ad2 = _pre_call(h1, W2.astype(jnp.float32), am2, bm2, F)
    Q0, Q1 = _edge1(S2, Ad2, src, dst, zeros_pk)
    return _post_call(S2, Ad2, Q0, Q1, b2.reshape(1, F), R2)


# speedup vs baseline: 99.3635x; 1.1264x over previous
import functools

import jax
import jax.numpy as jnp
from jax import lax
from jax.experimental import pallas as pl
from jax.experimental.pallas import tpu as pltpu
from jax.experimental.pallas import tpu_sc as plsc

N = 10000
E = 320000
D_IN = 128
F = 64
AW = 16
PK = AW + F

NC = 2
NS = 16
NW = NC * NS
EPW = E // NW
CHUNK = 80
NCHUNK = EPW // CHUNK
NP = 10240
RPT = NP // NS

BLK = 400
GRID = N // BLK


def _pre_body(x_ref, w_ref, am_ref, bm_ref, s_out, ad_out):
    h = jnp.dot(x_ref[:], w_ref[:], preferred_element_type=jnp.float32)
    asrc = jnp.dot(h, am_ref[:], preferred_element_type=jnp.float32)
    adst = jnp.dot(h, bm_ref[:], preferred_element_type=jnp.float32)
    s_out[:] = jnp.concatenate([asrc, h], axis=1)
    ad_out[:] = adst


def _pre_call(x, W, am, bm, d_in):
    return pl.pallas_call(
        _pre_body,
        grid=(GRID,),
        in_specs=[
            pl.BlockSpec((BLK, d_in), lambda i: (i, 0)),
            pl.BlockSpec((d_in, F), lambda i: (0, 0)),
            pl.BlockSpec((F, AW), lambda i: (0, 0)),
            pl.BlockSpec((F, AW), lambda i: (0, 0)),
        ],
        out_specs=[
            pl.BlockSpec((BLK, PK), lambda i: (i, 0)),
            pl.BlockSpec((BLK, AW), lambda i: (i, 0)),
        ],
        out_shape=[
            jax.ShapeDtypeStruct((N, PK), jnp.float32),
            jax.ShapeDtypeStruct((N, AW), jnp.float32),
        ],
    )(x, W, am, bm)


def _post_body(s_ref, ad_ref, p0_ref, p1_ref, b_ref, r_ref, o_ref):
    S = s_ref[:]
    a16 = S[:, 0:AW]
    h = S[:, AW:PK]
    al = a16 + ad_ref[:]
    ps = jnp.exp(jnp.maximum(al, 0.2 * al))
    den16 = ps + p0_ref[:, 0:AW] + p1_ref[:, 0:AW]
    R = r_ref[:]
    msg = (jnp.dot(ps, R, preferred_element_type=jnp.float32) * h
           + p0_ref[:, AW:PK] + p1_ref[:, AW:PK])
    den = jnp.dot(den16, R, preferred_element_type=jnp.float32)
    out = msg / (den + 1e-16) + b_ref[:]
    o_ref[:] = jnp.where(out > 0, out, jnp.exp(jnp.minimum(out, 0.0)) - 1.0)


def _post_call(S, Ad, P0, P1, b, R):
    return pl.pallas_call(
        _post_body,
        grid=(GRID,),
        in_specs=[
            pl.BlockSpec((BLK, PK), lambda i: (i, 0)),
            pl.BlockSpec((BLK, AW), lambda i: (i, 0)),
            pl.BlockSpec((BLK, PK), lambda i: (i, 0)),
            pl.BlockSpec((BLK, PK), lambda i: (i, 0)),
            pl.BlockSpec((1, F), lambda i: (0, 0)),
            pl.BlockSpec((AW, F), lambda i: (0, 0)),
        ],
        out_specs=pl.BlockSpec((BLK, F), lambda i: (i, 0)),
        out_shape=jax.ShapeDtypeStruct((N, F), jnp.float32),
    )(S, Ad, P0, P1, b, R)


def _vgather(vec, idx):
    dn = lax.GatherDimensionNumbers(
        offset_dims=(), collapsed_slice_dims=(0,), start_index_map=(0,))
    return lax.gather(vec, idx[:, None], dn, slice_sizes=(1,),
                      mode=lax.GatherScatterMode.PROMISE_IN_BOUNDS)


def _make_edge(heads):
    mesh = plsc.VectorSubcoreMesh(core_axis_name="c", subcore_axis_name="s")

    @functools.partial(
        pl.kernel,
        mesh=mesh,
        out_type=[
            jax.ShapeDtypeStruct((NP, PK), jnp.float32),
            jax.ShapeDtypeStruct((NP, PK), jnp.float32),
        ],
        scratch_types=[
            pltpu.VMEM((NCHUNK, CHUNK), jnp.int32),
            pltpu.VMEM((NCHUNK, CHUNK), jnp.int32),
            pltpu.VMEM((CHUNK, PK), jnp.float32),
            pltpu.VMEM((CHUNK, PK), jnp.float32),
            pltpu.VMEM((CHUNK, AW), jnp.float32),
            pltpu.VMEM((CHUNK, AW), jnp.float32),
            pltpu.VMEM((CHUNK, PK), jnp.float32),
            pltpu.VMEM((CHUNK, PK), jnp.float32),
            pltpu.VMEM((CHUNK,), jnp.int32),
            pltpu.VMEM_SHARED((NP, PK), jnp.float32),
            pltpu.SemaphoreType.DMA,
            pltpu.SemaphoreType.DMA,
            pltpu.SemaphoreType.DMA,
            pltpu.SemaphoreType.DMA,
            pltpu.SemaphoreType.DMA,
            pltpu.SemaphoreType.DMA,
        ],
        compiler_params=pltpu.CompilerParams(use_tc_tiling_on_sc=False),
    )
    def edge_kernel(s_hbm, ad_hbm, src_hbm, dst_hbm, zero_hbm,
                    out0, out1, src_all, dst_all, srow0, srow1,
                    arow0, arow1, sbuf0, sbuf1, padidx, acc,
                    ga0, gb0, ga1, gb1, ss0, ss1):
        c = lax.axis_index("c")
        s = lax.axis_index("s")
        wid = s * NC + c
        pltpu.sync_copy(zero_hbm.at[pl.ds(s * RPT, RPT)],
                        acc.at[pl.ds(s * RPT, RPT)])
        pltpu.sync_copy(src_hbm.at[pl.ds(wid * NCHUNK, NCHUNK)], src_all)
        pltpu.sync_copy(dst_hbm.at[pl.ds(wid * NCHUNK, NCHUNK)], dst_all)
        plsc.subcore_barrier()
        lane = lax.iota(jnp.int32, 16)

        for m in range(CHUNK // 16):
            padidx[pl.ds(16 * m, 16)] = jnp.full((16,), N, dtype=jnp.int32)
        pltpu.async_copy(sbuf0, acc.at[padidx], ss0, add=True)
        pltpu.async_copy(sbuf1, acc.at[padidx], ss1, add=True)

        def start(i, srow, arow, g1, g2):
            pltpu.async_copy(s_hbm.at[src_all.at[i]], srow, g1)
            pltpu.async_copy(ad_hbm.at[dst_all.at[i]], arow, g2)

        def process(i, srow, arow, sbuf, g1, g2, ss):
            pltpu.make_async_copy(s_hbm.at[src_all.at[i]], srow, g1).wait()
            pltpu.make_async_copy(ad_hbm.at[dst_all.at[i]], arow, g2).wait()
            pltpu.make_async_copy(sbuf, acc.at[dst_all.at[i]], ss).wait()

            def edge_body(e, c2):
                va = srow[e, pl.ds(0, AW)]
                vb = arow[e, pl.ds(0, AW)]
                al = va + vb
                p = jnp.exp(jnp.maximum(al, 0.2 * al))
                sbuf[e, pl.ds(0, AW)] = p
                for k in range(1, 5):
                    if heads == 8:
                        col_idx = (lane + (16 * (k - 1))) >> 3
                    else:
                        col_idx = lane * 0
                    hv = srow[e, pl.ds(16 * k, 16)]
                    pm = _vgather(p, col_idx)
                    sbuf[e, pl.ds(16 * k, 16)] = hv * pm
                return c2

            lax.fori_loop(0, CHUNK, edge_body, 0)
            pltpu.async_copy(sbuf, acc.at[dst_all.at[i]], ss, add=True)

        start(0, srow0, arow0, ga0, gb0)

        def pair_body(j, carry):
            start(2 * j + 1, srow1, arow1, ga1, gb1)
            process(2 * j, srow0, arow0, sbuf0, ga0, gb0, ss0)
            start(2 * j + 2, srow0, arow0, ga0, gb0)
            process(2 * j + 1, srow1, arow1, sbuf1, ga1, gb1, ss1)
            return carry

        lax.fori_loop(0, (NCHUNK - 1) // 2, pair_body, 0)
        process(NCHUNK - 1, srow0, arow0, sbuf0, ga0, gb0, ss0)
        pltpu.make_async_copy(sbuf0, acc.at[padidx], ss0).wait()
        pltpu.make_async_copy(sbuf1, acc.at[padidx], ss1).wait()
        plsc.subcore_barrier()

        @pl.when(c == 0)
        def _():
            pltpu.sync_copy(acc.at[pl.ds(s * RPT, RPT)],
                            out0.at[pl.ds(s * RPT, RPT)])

        @pl.when(c == 1)
        def _():
            pltpu.sync_copy(acc.at[pl.ds(s * RPT, RPT)],
                            out1.at[pl.ds(s * RPT, RPT)])

    return edge_kernel


_edge8 = _make_edge(8)
_edge1 = _make_edge(1)


def _placement(att, heads, ch):
    a = att.reshape(heads * ch).astype(jnp.float32)
    rows = jnp.arange(F)
    return jnp.zeros((F, AW), jnp.float32).at[rows, rows // ch].set(a)


def _rep_matrix(ch):
    col_head = jnp.arange(F) // ch
    return (col_head[None, :] == jnp.arange(AW)[:, None]).astype(jnp.float32)


def kernel(x, edge_index, W1, att_src1, att_dst1, b1,
           W2, att_src2, att_dst2, b2):
    src = edge_index[0].astype(jnp.int32).reshape(E // CHUNK, CHUNK)
    dst = edge_index[1].astype(jnp.int32).reshape(E // CHUNK, CHUNK)
    zeros_pk = jnp.zeros((NP, PK), jnp.float32)
    am1 = _placement(att_src1, 8, 8)
    bm1 = _placement(att_dst1, 8, 8)
    am2 = _placement(att_src2, 1, 64)
    bm2 = _placement(att_dst2, 1, 64)
    R1 = _rep_matrix(8)
    R2 = _rep_matrix(64)

    S1, Ad1 = _pre_call(x, W1.astype(jnp.float32), am1, bm1, D_IN)
    P0, P1 = _edge8(S1, Ad1, src, dst, zeros_pk)
    h1 = _post_call(S1, Ad1, P0, P1, b1.reshape(1, F), R1)

    S2, Ad2 = _pre_call(h1, W2.astype(jnp.float32), am2, bm2, F)
    Q0, Q1 = _edge1(S2, Ad2, src, dst, zeros_pk)
    return _post_call(S2, Ad2, Q0, Q1, b2.reshape(1, F), R2)


# speedup vs baseline: 146.2375x; 1.4717x over previous
import functools

import jax
import jax.numpy as jnp
from jax import lax
from jax.experimental import pallas as pl
from jax.experimental.pallas import tpu as pltpu
from jax.experimental.pallas import tpu_sc as plsc

N = 10000
E = 320000
D_IN = 128
F = 64
AW = 16
PK = AW + F

NC = 2
NS = 16
NW = NC * NS
EPW = E // NW
CHUNK = 80
NCHUNK = EPW // CHUNK
NP = 10240
RPT = NP // NS

BLK = 400
GRID = N // BLK


def _pre_body(x_ref, w_ref, am_ref, bm_ref, s_out, ad_out):
    h = jnp.dot(x_ref[:], w_ref[:], preferred_element_type=jnp.float32)
    asrc = jnp.dot(h, am_ref[:], preferred_element_type=jnp.float32)
    adst = jnp.dot(h, bm_ref[:], preferred_element_type=jnp.float32)
    s_out[:] = jnp.concatenate([asrc, h], axis=1)
    ad_out[:] = adst


def _pre_call(x, W, am, bm, d_in):
    return pl.pallas_call(
        _pre_body,
        grid=(GRID,),
        in_specs=[
            pl.BlockSpec((BLK, d_in), lambda i: (i, 0)),
            pl.BlockSpec((d_in, F), lambda i: (0, 0)),
            pl.BlockSpec((F, AW), lambda i: (0, 0)),
            pl.BlockSpec((F, AW), lambda i: (0, 0)),
        ],
        out_specs=[
            pl.BlockSpec((BLK, PK), lambda i: (i, 0)),
            pl.BlockSpec((BLK, AW), lambda i: (i, 0)),
        ],
        out_shape=[
            jax.ShapeDtypeStruct((N, PK), jnp.float32),
            jax.ShapeDtypeStruct((N, AW), jnp.float32),
        ],
    )(x, W, am, bm)


def _post_body(s_ref, ad_ref, p0_ref, p1_ref, b_ref, r_ref, o_ref):
    S = s_ref[:]
    a16 = S[:, 0:AW]
    h = S[:, AW:PK]
    al = a16 + ad_ref[:]
    ps = jnp.exp(jnp.maximum(al, 0.2 * al))
    den16 = ps + p0_ref[:, 0:AW] + p1_ref[:, 0:AW]
    R = r_ref[:]
    msg = (jnp.dot(ps, R, preferred_element_type=jnp.float32) * h
           + p0_ref[:, AW:PK] + p1_ref[:, AW:PK])
    den = jnp.dot(den16, R, preferred_element_type=jnp.float32)
    out = msg / (den + 1e-16) + b_ref[:]
    o_ref[:] = jnp.where(out > 0, out, jnp.exp(jnp.minimum(out, 0.0)) - 1.0)


def _post_call(S, Ad, P0, P1, b, R):
    return pl.pallas_call(
        _post_body,
        grid=(GRID,),
        in_specs=[
            pl.BlockSpec((BLK, PK), lambda i: (i, 0)),
            pl.BlockSpec((BLK, AW), lambda i: (i, 0)),
            pl.BlockSpec((BLK, PK), lambda i: (i, 0)),
            pl.BlockSpec((BLK, PK), lambda i: (i, 0)),
            pl.BlockSpec((1, F), lambda i: (0, 0)),
            pl.BlockSpec((AW, F), lambda i: (0, 0)),
        ],
        out_specs=pl.BlockSpec((BLK, F), lambda i: (i, 0)),
        out_shape=jax.ShapeDtypeStruct((N, F), jnp.float32),
    )(S, Ad, P0, P1, b, R)


def _vgather(vec, idx):
    dn = lax.GatherDimensionNumbers(
        offset_dims=(), collapsed_slice_dims=(0,), start_index_map=(0,))
    return lax.gather(vec, idx[:, None], dn, slice_sizes=(1,),
                      mode=lax.GatherScatterMode.PROMISE_IN_BOUNDS)


def _make_edge(heads):
    mesh = plsc.VectorSubcoreMesh(core_axis_name="c", subcore_axis_name="s")

    @functools.partial(
        pl.kernel,
        mesh=mesh,
        out_type=[
            jax.ShapeDtypeStruct((NP, PK), jnp.float32),
            jax.ShapeDtypeStruct((NP, PK), jnp.float32),
        ],
        scratch_types=[
            pltpu.VMEM((NCHUNK, CHUNK), jnp.int32),
            pltpu.VMEM((NCHUNK, CHUNK), jnp.int32),
            pltpu.VMEM((CHUNK, PK), jnp.float32),
            pltpu.VMEM((CHUNK, PK), jnp.float32),
            pltpu.VMEM((CHUNK, AW), jnp.float32),
            pltpu.VMEM((CHUNK, AW), jnp.float32),
            pltpu.VMEM((CHUNK, PK), jnp.float32),
            pltpu.VMEM((CHUNK, PK), jnp.float32),
            pltpu.VMEM((CHUNK,), jnp.int32),
            pltpu.VMEM_SHARED((NP, PK), jnp.float32),
            pltpu.SemaphoreType.DMA,
            pltpu.SemaphoreType.DMA,
            pltpu.SemaphoreType.DMA,
            pltpu.SemaphoreType.DMA,
            pltpu.SemaphoreType.DMA,
            pltpu.SemaphoreType.DMA,
        ],
        compiler_params=pltpu.CompilerParams(use_tc_tiling_on_sc=False),
    )
    def edge_kernel(s_hbm, ad_hbm, src_hbm, dst_hbm, zero_hbm,
                    out0, out1, src_all, dst_all, srow0, srow1,
                    arow0, arow1, sbuf0, sbuf1, padidx, acc,
                    ga0, gb0, ga1, gb1, ss0, ss1):
        c = lax.axis_index("c")
        s = lax.axis_index("s")
        wid = s * NC + c
        pltpu.sync_copy(zero_hbm.at[pl.ds(s * RPT, RPT)],
                        acc.at[pl.ds(s * RPT, RPT)])
        pltpu.sync_copy(src_hbm.at[pl.ds(wid * NCHUNK, NCHUNK)], src_all)
        pltpu.sync_copy(dst_hbm.at[pl.ds(wid * NCHUNK, NCHUNK)], dst_all)
        plsc.subcore_barrier()
        lane = lax.iota(jnp.int32, 16)

        for m in range(CHUNK // 16):
            padidx[pl.ds(16 * m, 16)] = jnp.full((16,), N, dtype=jnp.int32)
        pltpu.async_copy(sbuf0, acc.at[padidx], ss0, add=True)
        pltpu.async_copy(sbuf1, acc.at[padidx], ss1, add=True)

        def start(i, srow, arow, g1, g2):
            pltpu.async_copy(s_hbm.at[src_all.at[i]], srow, g1)
            pltpu.async_copy(ad_hbm.at[dst_all.at[i]], arow, g2)

        def process(i, srow, arow, sbuf, g1, g2, ss):
            pltpu.make_async_copy(s_hbm.at[src_all.at[i]], srow, g1).wait()
            pltpu.make_async_copy(ad_hbm.at[dst_all.at[i]], arow, g2).wait()
            pltpu.make_async_copy(sbuf, acc.at[dst_all.at[i]], ss).wait()

            @plsc.parallel_loop(0, CHUNK, unroll=8)
            def edge_body(e):
                va = srow[e, pl.ds(0, AW)]
                vb = arow[e, pl.ds(0, AW)]
                al = va + vb
                p = jnp.exp(jnp.maximum(al, 0.2 * al))
                sbuf[e, pl.ds(0, AW)] = p
                for k in range(1, 5):
                    if heads == 8:
                        col_idx = (lane + (16 * (k - 1))) >> 3
                    else:
                        col_idx = lane * 0
                    hv = srow[e, pl.ds(16 * k, 16)]
                    pm = _vgather(p, col_idx)
                    sbuf[e, pl.ds(16 * k, 16)] = hv * pm
            pltpu.async_copy(sbuf, acc.at[dst_all.at[i]], ss, add=True)

        start(0, srow0, arow0, ga0, gb0)

        def pair_body(j, carry):
            start(2 * j + 1, srow1, arow1, ga1, gb1)
            process(2 * j, srow0, arow0, sbuf0, ga0, gb0, ss0)
            start(2 * j + 2, srow0, arow0, ga0, gb0)
            process(2 * j + 1, srow1, arow1, sbuf1, ga1, gb1, ss1)
            return carry

        lax.fori_loop(0, (NCHUNK - 1) // 2, pair_body, 0)
        process(NCHUNK - 1, srow0, arow0, sbuf0, ga0, gb0, ss0)
        pltpu.make_async_copy(sbuf0, acc.at[padidx], ss0).wait()
        pltpu.make_async_copy(sbuf1, acc.at[padidx], ss1).wait()
        plsc.subcore_barrier()

        @pl.when(c == 0)
        def _():
            pltpu.sync_copy(acc.at[pl.ds(s * RPT, RPT)],
                            out0.at[pl.ds(s * RPT, RPT)])

        @pl.when(c == 1)
        def _():
            pltpu.sync_copy(acc.at[pl.ds(s * RPT, RPT)],
                            out1.at[pl.ds(s * RPT, RPT)])

    return edge_kernel


_edge8 = _make_edge(8)
_edge1 = _make_edge(1)


def _placement(att, heads, ch):
    a = att.reshape(heads * ch).astype(jnp.float32)
    rows = jnp.arange(F)
    return jnp.zeros((F, AW), jnp.float32).at[rows, rows // ch].set(a)


def _rep_matrix(ch):
    col_head = jnp.arange(F) // ch
    return (col_head[None, :] == jnp.arange(AW)[:, None]).astype(jnp.float32)


def kernel(x, edge_index, W1, att_src1, att_dst1, b1,
           W2, att_src2, att_dst2, b2):
    src = edge_index[0].astype(jnp.int32).reshape(E // CHUNK, CHUNK)
    dst = edge_index[1].astype(jnp.int32).reshape(E // CHUNK, CHUNK)
    zeros_pk = jnp.zeros((NP, PK), jnp.float32)
    am1 = _placement(att_src1, 8, 8)
    bm1 = _placement(att_dst1, 8, 8)
    am2 = _placement(att_src2, 1, 64)
    bm2 = _placement(att_dst2, 1, 64)
    R1 = _rep_matrix(8)
    R2 = _rep_matrix(64)

    S1, Ad1 = _pre_call(x, W1.astype(jnp.float32), am1, bm1, D_IN)
    P0, P1 = _edge8(S1, Ad1, src, dst, zeros_pk)
    h1 = _post_call(S1, Ad1, P0, P1, b1.reshape(1, F), R1)

    S2, Ad2 = _pre_call(h1, W2.astype(jnp.float32), am2, bm2, F)
    Q0, Q1 = _edge1(S2, Ad2, src, dst, zeros_pk)
    return _post_call(S2, Ad2, Q0, Q1, b2.reshape(1, F), R2)
